# Initial kernel scaffold; baseline (speedup 1.0000x reference)
import jax
import jax.numpy as jnp
from jax.experimental import pallas as pl


def kernel(inputs, user_embedding, user_bias, book_embedding, book_bias):
    raise NotImplementedError("write your pallas kernel here")


# speedup vs baseline: 1.6324x; 1.6324x over previous
import jax
import jax.numpy as jnp
from jax import lax
from jax.experimental import pallas as pl
from jax.experimental.pallas import tpu as pltpu
from jax.experimental.pallas import tpu_sc as plsc

B = 16384
D = 128
L = 16
NC = 2
NS = 16
NW = NC * NS
BPW = B // NW
CHUNK = 128
NCHUNK = BPW // CHUNK


def _sc_body(idx_u_hbm, idx_b_hbm, ue_hbm, ub_hbm, be_hbm, bb_hbm,
             part_out, ubg_out, bbg_out,
             idx_u_v, idx_b_v, ubuf, bbuf, ubias_v, bbias_v, acc_v,
             sem_u0, sem_u1, sem_b0, sem_b1, sem_ub, sem_bb):
    c = lax.axis_index("c")
    s = lax.axis_index("s")
    wid = s * NC + c

    pltpu.sync_copy(idx_u_hbm.at[wid], idx_u_v)
    pltpu.sync_copy(idx_b_hbm.at[wid], idx_b_v)

    ub_copies = [
        pltpu.async_copy(ub_hbm.at[idx_u_v.at[ci]], ubias_v.at[ci], sem_ub)
        for ci in range(NCHUNK)
    ]
    bb_copies = [
        pltpu.async_copy(bb_hbm.at[idx_b_v.at[ci]], bbias_v.at[ci], sem_bb)
        for ci in range(NCHUNK)
    ]

    sems_u = (sem_u0, sem_u1)
    sems_b = (sem_b0, sem_b1)

    def start(ci):
        slot = ci % 2
        cu = pltpu.async_copy(ue_hbm.at[idx_u_v.at[ci]], ubuf.at[slot], sems_u[slot])
        cb = pltpu.async_copy(be_hbm.at[idx_b_v.at[ci]], bbuf.at[slot], sems_b[slot])
        return cu, cb

    pend = start(0)
    acc = jnp.zeros((L,), jnp.float32)
    for ci in range(NCHUNK):
        nxt = start(ci + 1) if ci + 1 < NCHUNK else None
        pend[0].wait()
        pend[1].wait()
        slot = ci % 2

        def row_body(r, a, _slot=slot):
            for k in range(D // L):
                a = a + (ubuf[_slot, r, pl.ds(k * L, L)]
                         * bbuf[_slot, r, pl.ds(k * L, L)])
            return a

        acc = lax.fori_loop(0, CHUNK, row_body, acc)
        pend = nxt

    acc_v[...] = acc
    pltpu.sync_copy(acc_v, part_out.at[wid])

    for cpy in ub_copies:
        cpy.wait()
    for cpy in bb_copies:
        cpy.wait()
    pltpu.sync_copy(ubias_v, ubg_out.at[wid])
    pltpu.sync_copy(bbias_v, bbg_out.at[wid])


_sc_call = pl.kernel(
    _sc_body,
    mesh=plsc.VectorSubcoreMesh(core_axis_name="c", subcore_axis_name="s"),
    out_type=(
        jax.ShapeDtypeStruct((NW, L), jnp.float32),
        jax.ShapeDtypeStruct((NW, NCHUNK, CHUNK), jnp.float32),
        jax.ShapeDtypeStruct((NW, NCHUNK, CHUNK), jnp.float32),
    ),
    scratch_types=[
        pltpu.VMEM((NCHUNK, CHUNK), jnp.int32),
        pltpu.VMEM((NCHUNK, CHUNK), jnp.int32),
        pltpu.VMEM((2, CHUNK, D), jnp.float32),
        pltpu.VMEM((2, CHUNK, D), jnp.float32),
        pltpu.VMEM((NCHUNK, CHUNK), jnp.float32),
        pltpu.VMEM((NCHUNK, CHUNK), jnp.float32),
        pltpu.VMEM((L,), jnp.float32),
        pltpu.SemaphoreType.DMA,
        pltpu.SemaphoreType.DMA,
        pltpu.SemaphoreType.DMA,
        pltpu.SemaphoreType.DMA,
        pltpu.SemaphoreType.DMA,
        pltpu.SemaphoreType.DMA,
    ],
)


def _tc_body(part_ref, ub_ref, bb_ref, out_ref):
    dot = jnp.sum(part_ref[...])
    x = dot + ub_ref[...] + bb_ref[...]
    out_ref[...] = 1.0 / (1.0 + jnp.exp(-x))


def kernel(inputs, user_embedding, user_bias, book_embedding, book_bias):
    idx_u = inputs[:, 0].reshape(NW, NCHUNK, CHUNK)
    idx_b = inputs[:, 1].reshape(NW, NCHUNK, CHUNK)
    part, ubg, bbg = _sc_call(
        idx_u, idx_b,
        user_embedding, user_bias.reshape(-1),
        book_embedding, book_bias.reshape(-1),
    )
    out = pl.pallas_call(
        _tc_body,
        out_shape=jax.ShapeDtypeStruct((B // D, D), jnp.float32),
    )(part.reshape(NW * L // D, D), ubg.reshape(B // D, D), bbg.reshape(B // D, D))
    return out.reshape(B, 1)
